# baseline (speedup 1.0000x reference)
import jax
import jax.numpy as jnp
from jax.experimental import pallas as pl

_BB = 512


def _bcast_kernel(pe_ref, out_ref):
    out_ref[...] = jnp.broadcast_to(pe_ref[...], out_ref.shape)


def kernel(x, pe):
    batch, seq_len = x.shape
    max_len, d_model = pe.shape
    flat = seq_len * d_model
    pe_flat = pe.reshape(1, flat)

    out = pl.pallas_call(
        _bcast_kernel,
        grid=(batch // _BB,),
        in_specs=[pl.BlockSpec((1, flat), lambda i: (0, 0))],
        out_specs=pl.BlockSpec((_BB, flat), lambda i: (i, 0)),
        out_shape=jax.ShapeDtypeStruct((batch, flat), jnp.float32),
    )(pe_flat)
    return out.reshape(batch, seq_len, d_model)


# speedup vs baseline: 1.0026x; 1.0026x over previous
import jax
import jax.numpy as jnp
from jax.experimental import pallas as pl
from jax.experimental.pallas import tpu as pltpu

_BB = 512


def _bcast_kernel(pe_ref, out_ref, buf_ref, sems):
    n = out_ref.shape[0] // _BB
    buf_ref[...] = jnp.broadcast_to(pe_ref[...], buf_ref.shape)
    for i in range(n):
        pltpu.make_async_copy(
            buf_ref, out_ref.at[pl.ds(i * _BB, _BB), :], sems.at[i]
        ).start()
    for i in range(n):
        pltpu.make_async_copy(
            buf_ref, out_ref.at[pl.ds(i * _BB, _BB), :], sems.at[i]
        ).wait()


def kernel(x, pe):
    batch, seq_len = x.shape
    max_len, d_model = pe.shape
    flat = seq_len * d_model
    pe_flat = pe.reshape(1, flat)
    n = batch // _BB

    out = pl.pallas_call(
        _bcast_kernel,
        in_specs=[pl.BlockSpec(memory_space=pltpu.MemorySpace.VMEM)],
        out_specs=pl.BlockSpec(memory_space=pl.ANY),
        out_shape=jax.ShapeDtypeStruct((batch, flat), jnp.float32),
        scratch_shapes=[
            pltpu.VMEM((_BB, flat), jnp.float32),
            pltpu.SemaphoreType.DMA((n,)),
        ],
    )(pe_flat)
    return out.reshape(batch, seq_len, d_model)
